# Initial kernel scaffold; baseline (speedup 1.0000x reference)
import jax
import jax.numpy as jnp
from jax.experimental import pallas as pl


def kernel(hidden_states, gate_weight, expert_bias):
    raise NotImplementedError("write your pallas kernel here")


# speedup vs baseline: 1.7928x; 1.7928x over previous
import functools

import jax
import jax.numpy as jnp
from jax import lax
from jax.experimental import pallas as pl
from jax.experimental.pallas import tpu as pltpu

_NUM_EXPERTS = 64
_TOP_K = 8
_N_GROUP = 8
_TOPK_GROUP = 4
_EPG = _NUM_EXPERTS // _N_GROUP
_SCALE = 2.5
_BLOCK_T = 1024


def _gate_body(x_ref, wt_ref, bias_ref, logits_ref, idx_ref, w_ref):
    x = x_ref[...]
    wt = wt_ref[...]
    logits = jnp.dot(x, wt, preferred_element_type=jnp.float32)
    logits_ref[...] = logits

    scores = 1.0 / (1.0 + jnp.exp(-logits))
    s_r = scores + bias_ref[...]

    T = x.shape[0]
    lane = lax.broadcasted_iota(jnp.int32, (T, _NUM_EXPERTS), 1)
    gid = lane // _EPG
    neg = jnp.float32(-jnp.inf)

    gs = []
    for g in range(_N_GROUP):
        sg = jnp.where(gid == g, s_r, neg)
        m1 = jnp.max(sg, axis=1, keepdims=True)
        first = jnp.min(jnp.where(sg == m1, lane, _NUM_EXPERTS),
                        axis=1, keepdims=True)
        m2 = jnp.max(jnp.where(lane == first, neg, sg),
                     axis=1, keepdims=True)
        gs.append(m1 + m2)

    sel = []
    for g in range(_N_GROUP):
        rank = jnp.zeros((T, 1), dtype=jnp.int32)
        for h in range(_N_GROUP):
            if h == g:
                continue
            beats = gs[h] > gs[g]
            if h < g:
                beats = beats | (gs[h] == gs[g])
            rank = rank + beats.astype(jnp.int32)
        sel.append((rank < _TOPK_GROUP).astype(jnp.float32))

    gmask = jnp.zeros((T, _NUM_EXPERTS), dtype=jnp.float32)
    for g in range(_N_GROUP):
        gmask = jnp.where(gid == g,
                          jnp.broadcast_to(sel[g], (T, _NUM_EXPERTS)), gmask)
    masked = jnp.where(gmask > 0.5, s_r, neg)

    lane8 = lax.broadcasted_iota(jnp.int32, (T, _TOP_K), 1)
    out_idx = jnp.zeros((T, _TOP_K), dtype=jnp.int32)
    out_val = jnp.zeros((T, _TOP_K), dtype=jnp.float32)
    cur = masked
    for k in range(_TOP_K):
        m = jnp.max(cur, axis=1, keepdims=True)
        pick = jnp.min(jnp.where(cur == m, lane, _NUM_EXPERTS),
                       axis=1, keepdims=True)
        hit = lane == pick
        orig = jnp.max(jnp.where(hit, scores, neg), axis=1, keepdims=True)
        cur = jnp.where(hit, neg, cur)
        out_idx = jnp.where(lane8 == k,
                            jnp.broadcast_to(pick, (T, _TOP_K)), out_idx)
        out_val = jnp.where(lane8 == k,
                            jnp.broadcast_to(orig, (T, _TOP_K)), out_val)

    denom = jnp.sum(out_val, axis=1, keepdims=True) + 1e-20
    idx_ref[...] = out_idx
    w_ref[...] = out_val / denom * _SCALE


@functools.partial(jax.jit, static_argnames=())
def kernel(hidden_states, gate_weight, expert_bias):
    n_tokens, hidden = hidden_states.shape
    wt = gate_weight.T
    bias = expert_bias.reshape(1, _NUM_EXPERTS)
    grid = (n_tokens // _BLOCK_T,)
    out_shapes = (
        jax.ShapeDtypeStruct((n_tokens, _NUM_EXPERTS), jnp.float32),
        jax.ShapeDtypeStruct((n_tokens, _TOP_K), jnp.int32),
        jax.ShapeDtypeStruct((n_tokens, _TOP_K), jnp.float32),
    )
    logits, idx, w = pl.pallas_call(
        _gate_body,
        grid=grid,
        in_specs=[
            pl.BlockSpec((_BLOCK_T, hidden), lambda i: (i, 0)),
            pl.BlockSpec((hidden, _NUM_EXPERTS), lambda i: (0, 0)),
            pl.BlockSpec((1, _NUM_EXPERTS), lambda i: (0, 0)),
        ],
        out_specs=(
            pl.BlockSpec((_BLOCK_T, _NUM_EXPERTS), lambda i: (i, 0)),
            pl.BlockSpec((_BLOCK_T, _TOP_K), lambda i: (i, 0)),
            pl.BlockSpec((_BLOCK_T, _TOP_K), lambda i: (i, 0)),
        ),
        out_shape=out_shapes,
        compiler_params=pltpu.CompilerParams(
            dimension_semantics=("parallel",),
        ),
    )(hidden_states, wt, bias)
    return (idx, w, logits)


# speedup vs baseline: 10.0146x; 5.5860x over previous
import functools

import jax
import jax.numpy as jnp
from jax import lax
from jax.experimental import pallas as pl
from jax.experimental.pallas import tpu as pltpu

_NUM_EXPERTS = 64
_TOP_K = 8
_N_GROUP = 8
_TOPK_GROUP = 4
_EPG = _NUM_EXPERTS // _N_GROUP
_SCALE = 2.5
_BLOCK_T = 1024


def _gate_body(x_ref, w_ref, wt_ref, bias_ref, logits_ref, idx_ref, wout_ref):
    x = x_ref[...]
    logits = jnp.dot(x, wt_ref[...], preferred_element_type=jnp.float32)
    logits_ref[...] = logits

    logits_t = lax.dot_general(w_ref[...], x, (((1,), (1,)), ((), ())),
                               preferred_element_type=jnp.float32)
    scores = 1.0 / (1.0 + jnp.exp(-logits_t))
    s_r = scores + bias_ref[...]

    T = x.shape[0]
    neg = jnp.float32(-jnp.inf)
    sub8 = lax.broadcasted_iota(jnp.int32, (_EPG, T), 0)

    gs = []
    for g in range(_N_GROUP):
        band = s_r[g * _EPG:(g + 1) * _EPG, :]
        m1 = jnp.max(band, axis=0, keepdims=True)
        first = jnp.min(jnp.where(band == m1, sub8, _EPG),
                        axis=0, keepdims=True)
        m2 = jnp.max(jnp.where(sub8 == first, neg, band),
                     axis=0, keepdims=True)
        gs.append(m1 + m2)

    sel = []
    for g in range(_N_GROUP):
        rank = jnp.zeros((1, T), dtype=jnp.int32)
        for h in range(_N_GROUP):
            if h == g:
                continue
            beats = gs[h] > gs[g]
            if h < g:
                beats = beats | (gs[h] == gs[g])
            rank = rank + beats.astype(jnp.int32)
        sel.append(rank < _TOPK_GROUP)

    bands = []
    for g in range(_N_GROUP):
        keep = jnp.broadcast_to(sel[g], (_EPG, T))
        bands.append(jnp.where(keep, s_r[g * _EPG:(g + 1) * _EPG, :], neg))
    masked = jnp.concatenate(bands, axis=0)

    sub64 = lax.broadcasted_iota(jnp.int32, (_NUM_EXPERTS, T), 0)
    idx_rows = []
    val_rows = []
    cur = masked
    for k in range(_TOP_K):
        m = jnp.max(cur, axis=0, keepdims=True)
        pick = jnp.min(jnp.where(cur == m, sub64, _NUM_EXPERTS),
                       axis=0, keepdims=True)
        hit = sub64 == pick
        orig = jnp.max(jnp.where(hit, scores, neg), axis=0, keepdims=True)
        cur = jnp.where(hit, neg, cur)
        idx_rows.append(pick)
        val_rows.append(orig)

    idx_t = jnp.concatenate(idx_rows, axis=0)
    val_t = jnp.concatenate(val_rows, axis=0)
    denom = jnp.sum(val_t, axis=0, keepdims=True) + 1e-20
    idx_ref[...] = idx_t
    wout_ref[...] = val_t / denom * _SCALE


@functools.partial(jax.jit, static_argnames=())
def kernel(hidden_states, gate_weight, expert_bias):
    n_tokens, hidden = hidden_states.shape
    wt = gate_weight.T
    bias = expert_bias.reshape(_NUM_EXPERTS, 1)
    grid = (n_tokens // _BLOCK_T,)
    out_shapes = (
        jax.ShapeDtypeStruct((n_tokens, _NUM_EXPERTS), jnp.float32),
        jax.ShapeDtypeStruct((_TOP_K, n_tokens), jnp.int32),
        jax.ShapeDtypeStruct((_TOP_K, n_tokens), jnp.float32),
    )
    logits, idx_t, w_t = pl.pallas_call(
        _gate_body,
        grid=grid,
        in_specs=[
            pl.BlockSpec((_BLOCK_T, hidden), lambda i: (i, 0)),
            pl.BlockSpec((_NUM_EXPERTS, hidden), lambda i: (0, 0)),
            pl.BlockSpec((hidden, _NUM_EXPERTS), lambda i: (0, 0)),
            pl.BlockSpec((_NUM_EXPERTS, 1), lambda i: (0, 0)),
        ],
        out_specs=(
            pl.BlockSpec((_BLOCK_T, _NUM_EXPERTS), lambda i: (i, 0)),
            pl.BlockSpec((_TOP_K, _BLOCK_T), lambda i: (0, i)),
            pl.BlockSpec((_TOP_K, _BLOCK_T), lambda i: (0, i)),
        ),
        out_shape=out_shapes,
        compiler_params=pltpu.CompilerParams(
            dimension_semantics=("parallel",),
        ),
    )(hidden_states, gate_weight, wt, bias)
    return (idx_t.T, w_t.T, logits)


# speedup vs baseline: 10.7372x; 1.0722x over previous
import functools

import jax
import jax.numpy as jnp
from jax import lax
from jax.experimental import pallas as pl
from jax.experimental.pallas import tpu as pltpu

_NUM_EXPERTS = 64
_TOP_K = 8
_N_GROUP = 8
_TOPK_GROUP = 4
_EPG = _NUM_EXPERTS // _N_GROUP
_SCALE = 2.5
_BLOCK_T = 1024


def _gate_body(x_ref, w_ref, bias_ref, logits_ref, idx_ref, wout_ref):
    x = x_ref[...]
    logits_t = lax.dot_general(w_ref[...], x, (((1,), (1,)), ((), ())),
                               preferred_element_type=jnp.float32)
    logits_ref[...] = logits_t.T

    scores = 1.0 / (1.0 + jnp.exp(-logits_t))
    s_r = scores + bias_ref[...]

    T = x.shape[0]
    neg = jnp.float32(-jnp.inf)

    gs = []
    for g in range(_N_GROUP):
        band = s_r[g * _EPG:(g + 1) * _EPG, :]
        m1 = jnp.max(band, axis=0, keepdims=True)
        m2 = jnp.max(jnp.where(band == m1, neg, band),
                     axis=0, keepdims=True)
        gs.append(m1 + m2)
    gstack = jnp.concatenate(gs, axis=0)

    rank = jnp.zeros((_N_GROUP, T), dtype=jnp.int32)
    for r in range(1, _N_GROUP):
        rot = jnp.roll(gstack, -r, axis=0)
        beats = rot > gstack
        tie_rows = (lax.broadcasted_iota(jnp.int32, (_N_GROUP, T), 0)
                    >= _N_GROUP - r)
        beats = beats | ((rot == gstack) & tie_rows)
        rank = rank + beats.astype(jnp.int32)
    selg = rank < _TOPK_GROUP

    bands = []
    for g in range(_N_GROUP):
        keep = jnp.broadcast_to(selg[g:g + 1, :], (_EPG, T))
        bands.append(jnp.where(keep, s_r[g * _EPG:(g + 1) * _EPG, :], neg))
    masked = jnp.concatenate(bands, axis=0)

    sub64f = lax.broadcasted_iota(
        jnp.int32, (_NUM_EXPERTS, T), 0).astype(jnp.float32)
    combo = sub64f + scores
    idx_rows = []
    val_rows = []
    cur = masked
    for k in range(_TOP_K):
        m = jnp.max(cur, axis=0, keepdims=True)
        hit = cur == m
        c = jnp.max(jnp.where(hit, combo, neg), axis=0, keepdims=True)
        idxf = jnp.floor(c)
        idx_rows.append(idxf.astype(jnp.int32))
        val_rows.append(c - idxf)
        cur = jnp.where(hit, neg, cur)

    idx_t = jnp.concatenate(idx_rows, axis=0)
    val_t = jnp.concatenate(val_rows, axis=0)
    denom = jnp.sum(val_t, axis=0, keepdims=True) + 1e-20
    idx_ref[...] = idx_t
    wout_ref[...] = val_t / denom * _SCALE


@functools.partial(jax.jit, static_argnames=())
def kernel(hidden_states, gate_weight, expert_bias):
    n_tokens, hidden = hidden_states.shape
    bias = expert_bias.reshape(_NUM_EXPERTS, 1)
    grid = (n_tokens // _BLOCK_T,)
    out_shapes = (
        jax.ShapeDtypeStruct((n_tokens, _NUM_EXPERTS), jnp.float32),
        jax.ShapeDtypeStruct((_TOP_K, n_tokens), jnp.int32),
        jax.ShapeDtypeStruct((_TOP_K, n_tokens), jnp.float32),
    )
    logits, idx_t, w_t = pl.pallas_call(
        _gate_body,
        grid=grid,
        in_specs=[
            pl.BlockSpec((_BLOCK_T, hidden), lambda i: (i, 0)),
            pl.BlockSpec((_NUM_EXPERTS, hidden), lambda i: (0, 0)),
            pl.BlockSpec((_NUM_EXPERTS, 1), lambda i: (0, 0)),
        ],
        out_specs=(
            pl.BlockSpec((_BLOCK_T, _NUM_EXPERTS), lambda i: (i, 0)),
            pl.BlockSpec((_TOP_K, _BLOCK_T), lambda i: (0, i)),
            pl.BlockSpec((_TOP_K, _BLOCK_T), lambda i: (0, i)),
        ),
        out_shape=out_shapes,
        compiler_params=pltpu.CompilerParams(
            dimension_semantics=("parallel",),
        ),
    )(hidden_states, gate_weight, bias)
    return (idx_t.T, w_t.T, logits)
